# baseline (speedup 1.0000x reference)
import math

import numpy as np

import jax
import jax.numpy as jnp
from jax import lax
from jax.experimental import pallas as pl
from jax.experimental.pallas import tpu as pltpu
from jax.sharding import Mesh, PartitionSpec as P

try:
    from jax.experimental.shard_map import shard_map
except ImportError:
    from jax.shard_map import shard_map


def _round_up(v, m):
    return ((v + m - 1) // m) * m


def _linear_row_kernel(x_ref, w_ref, b_ref, o_ref, wbf_ref):
    @pl.when(pl.program_id(0) == 0)
    def _():
        wbf_ref[...] = w_ref[...].astype(jnp.bfloat16)

    xb = x_ref[...].astype(jnp.bfloat16)
    acc = lax.dot_general(
        xb, wbf_ref[...],
        dimension_numbers=(((1,), (1,)), ((), ())),
        preferred_element_type=jnp.float32)
    o_ref[...] = acc + b_ref[...]


def _linear_pallas(x2d, w, b2d, tm):
    rows, Kp = x2d.shape
    Np = w.shape[0]
    return pl.pallas_call(
        _linear_row_kernel,
        out_shape=jax.ShapeDtypeStruct((rows, Np), jnp.float32),
        grid=(rows // tm,),
        in_specs=[
            pl.BlockSpec((tm, Kp), lambda i: (i, 0)),
            pl.BlockSpec((Np, Kp), lambda i: (0, 0)),
            pl.BlockSpec((1, Np), lambda i: (0, 0)),
        ],
        out_specs=pl.BlockSpec((tm, Np), lambda i: (i, 0)),
        scratch_shapes=[pltpu.VMEM((Np, Kp), jnp.bfloat16)],
        compiler_params=pltpu.CompilerParams(
            dimension_semantics=("arbitrary",)),
    )(x2d, w, b2d)


def kernel(x, weight, bias):
    *lead, K = x.shape
    N, Kw = weight.shape
    assert Kw == K
    M = int(math.prod(lead)) if lead else 1

    x2d = x.reshape(M, K)

    tm = min(256, _round_up(M, 8))
    Mp, Np, Kp = _round_up(M, tm), _round_up(N, 128), _round_up(K, 128)

    devs = jax.devices()
    n_shards = 2 if (len(devs) >= 2 and Mp % (2 * tm) == 0) else 1
    Mp = _round_up(Mp, n_shards * tm)

    if (Mp, Kp) != (M, K):
        x2d = jnp.pad(x2d, ((0, Mp - M), (0, Kp - K)))
    w = weight
    if (Np, Kp) != (N, K):
        w = jnp.pad(w, ((0, Np - N), (0, Kp - K)))
    b = bias if Np == N else jnp.pad(bias, ((0, Np - N),))
    b2d = b.reshape(1, Np).astype(jnp.float32)

    if n_shards == 1:
        out = _linear_pallas(x2d, w, b2d, tm)
    else:
        mesh = Mesh(np.array(devs[:n_shards]), ("d",))
        out = shard_map(
            lambda xs, ws, bs: _linear_pallas(xs, ws, bs, tm),
            mesh=mesh,
            in_specs=(P("d", None), P(None, None), P(None, None)),
            out_specs=P("d", None),
            check_rep=False,
        )(x2d, w, b2d)

    out = out[:M, :N].astype(x.dtype)
    return out.reshape(*lead, N)


# speedup vs baseline: 9.3869x; 9.3869x over previous
import math

import jax
import jax.numpy as jnp
from jax import lax
from jax.experimental import pallas as pl
from jax.experimental.pallas import tpu as pltpu


def _round_up(v, m):
    return ((v + m - 1) // m) * m


def _linear_row_kernel(x_ref, w_ref, b_ref, o_ref, wbf_ref):
    @pl.when(pl.program_id(0) == 0)
    def _():
        wbf_ref[...] = w_ref[...].astype(jnp.bfloat16)

    xb = x_ref[...].astype(jnp.bfloat16)
    acc = lax.dot_general(
        xb, wbf_ref[...],
        dimension_numbers=(((1,), (1,)), ((), ())),
        preferred_element_type=jnp.float32)
    o_ref[...] = acc + b_ref[...]


def _linear_pallas(x2d, w, b2d, tm):
    rows, Kp = x2d.shape
    Np = w.shape[0]
    return pl.pallas_call(
        _linear_row_kernel,
        out_shape=jax.ShapeDtypeStruct((rows, Np), jnp.float32),
        grid=(rows // tm,),
        in_specs=[
            pl.BlockSpec((tm, Kp), lambda i: (i, 0)),
            pl.BlockSpec((Np, Kp), lambda i: (0, 0)),
            pl.BlockSpec((1, Np), lambda i: (0, 0)),
        ],
        out_specs=pl.BlockSpec((tm, Np), lambda i: (i, 0)),
        scratch_shapes=[pltpu.VMEM((Np, Kp), jnp.bfloat16)],
        compiler_params=pltpu.CompilerParams(
            dimension_semantics=("arbitrary",)),
    )(x2d, w, b2d)


def kernel(x, weight, bias):
    *lead, K = x.shape
    N, Kw = weight.shape
    assert Kw == K
    M = int(math.prod(lead)) if lead else 1

    x2d = x.reshape(M, K)

    tm = min(512, _round_up(M, 8))
    Mp, Np, Kp = _round_up(M, tm), _round_up(N, 128), _round_up(K, 128)

    if (Mp, Kp) != (M, K):
        x2d = jnp.pad(x2d, ((0, Mp - M), (0, Kp - K)))
    w = weight
    if (Np, Kp) != (N, K):
        w = jnp.pad(w, ((0, Np - N), (0, Kp - K)))
    b = bias if Np == N else jnp.pad(bias, ((0, Np - N),))
    b2d = b.reshape(1, Np).astype(jnp.float32)

    out = _linear_pallas(x2d, w, b2d, tm)

    out = out[:M, :N].astype(x.dtype)
    return out.reshape(*lead, N)


# speedup vs baseline: 9.3951x; 1.0009x over previous
import math

import jax
import jax.numpy as jnp
from jax import lax
from jax.experimental import pallas as pl
from jax.experimental.pallas import tpu as pltpu


def _round_up(v, m):
    return ((v + m - 1) // m) * m


def _linear_row_kernel(xa_ref, xb_ref, w_ref, b_ref, o_ref, wbf_ref):
    @pl.when(pl.program_id(0) == 0)
    def _():
        wbf_ref[...] = w_ref[...].astype(jnp.bfloat16)

    kh = xa_ref.shape[1]
    xa = xa_ref[...].astype(jnp.bfloat16)
    xb = xb_ref[...].astype(jnp.bfloat16)
    acc = lax.dot_general(
        xa, wbf_ref[:, :kh],
        dimension_numbers=(((1,), (1,)), ((), ())),
        preferred_element_type=jnp.float32)
    acc += lax.dot_general(
        xb, wbf_ref[:, kh:],
        dimension_numbers=(((1,), (1,)), ((), ())),
        preferred_element_type=jnp.float32)
    o_ref[...] = acc + b_ref[...]


def _linear_pallas(x2d, w, b2d, tm):
    rows, Kp = x2d.shape
    Np = w.shape[0]
    return pl.pallas_call(
        _linear_row_kernel,
        out_shape=jax.ShapeDtypeStruct((rows, Np), jnp.float32),
        grid=(rows // tm,),
        in_specs=[
            pl.BlockSpec((tm, Kp // 2), lambda i: (i, 0)),
            pl.BlockSpec((tm, Kp // 2), lambda i: (i, 1)),
            pl.BlockSpec((Np, Kp), lambda i: (0, 0)),
            pl.BlockSpec((1, Np), lambda i: (0, 0)),
        ],
        out_specs=pl.BlockSpec((tm, Np), lambda i: (i, 0)),
        scratch_shapes=[pltpu.VMEM((Np, Kp), jnp.bfloat16)],
        compiler_params=pltpu.CompilerParams(
            dimension_semantics=("arbitrary",)),
    )(x2d, x2d, w, b2d)


def kernel(x, weight, bias):
    *lead, K = x.shape
    N, Kw = weight.shape
    assert Kw == K
    M = int(math.prod(lead)) if lead else 1

    x2d = x.reshape(M, K)

    tm = min(512, _round_up(M, 8))
    Mp, Np, Kp = _round_up(M, tm), _round_up(N, 128), _round_up(K, 128)

    if (Mp, Kp) != (M, K):
        x2d = jnp.pad(x2d, ((0, Mp - M), (0, Kp - K)))
    w = weight
    if (Np, Kp) != (N, K):
        w = jnp.pad(w, ((0, Np - N), (0, Kp - K)))
    b = bias if Np == N else jnp.pad(bias, ((0, Np - N),))
    b2d = b.reshape(1, Np).astype(jnp.float32)

    out = _linear_pallas(x2d, w, b2d, tm)

    out = out[:M, :N].astype(x.dtype)
    return out.reshape(*lead, N)


# speedup vs baseline: 9.4457x; 1.0054x over previous
import math

import jax
import jax.numpy as jnp
from jax import lax
from jax.experimental import pallas as pl
from jax.experimental.pallas import tpu as pltpu


def _round_up(v, m):
    return ((v + m - 1) // m) * m


def _linear_row_kernel(x_ref, w_ref, b_ref, o_ref, wbf_ref):
    @pl.when(pl.program_id(0) == 0)
    def _():
        wbf_ref[...] = w_ref[...].astype(jnp.bfloat16)

    xb = x_ref[...].astype(jnp.bfloat16)
    acc = lax.dot_general(
        xb, wbf_ref[...],
        dimension_numbers=(((1,), (1,)), ((), ())),
        preferred_element_type=jnp.float32)
    o_ref[...] = acc + b_ref[...]


def _linear_pallas(x2d, w, b2d, tm):
    rows, Kp = x2d.shape
    Np = w.shape[0]
    return pl.pallas_call(
        _linear_row_kernel,
        out_shape=jax.ShapeDtypeStruct((rows, Np), jnp.float32),
        grid=(rows // tm,),
        in_specs=[
            pl.BlockSpec((tm, Kp), lambda i: (i, 0)),
            pl.BlockSpec((Np, Kp), lambda i: (0, 0)),
            pl.BlockSpec((1, Np), lambda i: (0, 0)),
        ],
        out_specs=pl.BlockSpec((tm, Np), lambda i: (i, 0)),
        scratch_shapes=[pltpu.VMEM((Np, Kp), jnp.bfloat16)],
        compiler_params=pltpu.CompilerParams(
            dimension_semantics=("arbitrary",)),
    )(x2d, w, b2d)


def kernel(x, weight, bias):
    *lead, K = x.shape
    N, Kw = weight.shape
    assert Kw == K
    M = int(math.prod(lead)) if lead else 1

    x2d = x.reshape(M, K)

    tm = min(512, _round_up(M, 8))
    Mp, Np, Kp = _round_up(M, tm), _round_up(N, 128), _round_up(K, 128)

    if (Mp, Kp) != (M, K):
        x2d = jnp.pad(x2d, ((0, Mp - M), (0, Kp - K)))
    w = weight
    if (Np, Kp) != (N, K):
        w = jnp.pad(w, ((0, Np - N), (0, Kp - K)))
    b = bias if Np == N else jnp.pad(bias, ((0, Np - N),))
    b2d = b.reshape(1, Np).astype(jnp.float32)

    out = _linear_pallas(x2d, w, b2d, tm)

    out = out[:M, :N].astype(x.dtype)
    return out.reshape(*lead, N)
